# baseline (speedup 1.0000x reference)
import functools

import jax
import jax.numpy as jnp
from jax import lax
from jax.experimental import pallas as pl
from jax.experimental.pallas import tpu as pltpu
from jax.experimental.pallas import tpu_sc as plsc

DIM = 64
RPB = 128
K = 5
NB = 2
NC = 2
NS = 16
NW = NC * NS
PACK_C = 2048


def _pack_body(t2_ref, z_ref, scr):
    t = t2_ref[...].T
    scr[:, 0:64] = t
    ev = scr[0::2, :]
    od = scr[1::2, :]
    odr = pltpu.roll(od, 64, 1)
    lane = lax.broadcasted_iota(jnp.int32, (PACK_C // 2, 128), 1)
    z_ref[...] = jnp.where(lane < 64, ev, odr)


def _pack_table(t2):
    vocab = t2.shape[1]
    return pl.pallas_call(
        _pack_body,
        grid=(pl.cdiv(vocab, PACK_C),),
        in_specs=[pl.BlockSpec((DIM, PACK_C), lambda j: (0, j))],
        out_specs=pl.BlockSpec((PACK_C // 2, 128), lambda j: (j, 0)),
        out_shape=jax.ShapeDtypeStruct((vocab // 2, 128), jnp.float32),
        scratch_shapes=[pltpu.VMEM((PACK_C, 128), jnp.float32)],
    )(t2)


XW = 131


@jax.jit
def _fmt_out(o3):
    mesh = plsc.VectorSubcoreMesh(core_axis_name="c", subcore_axis_name="s")

    @functools.partial(
        pl.kernel,
        mesh=mesh,
        out_type=jax.ShapeDtypeStruct((200 * 8 * 32 * 8 * 128,), jnp.float32),
        scratch_types=[
            pltpu.VMEM((128, 1, XW), jnp.float32),
            pltpu.VMEM((8192,), jnp.float32),
        ],
        compiler_params=pltpu.CompilerParams(use_tc_tiling_on_sc=False,
                                             needs_layout_passes=False),
    )
    def fmt_kernel(o3_hbm, f_hbm, xbuf, obuf):
        w = lax.axis_index("s") * NC + lax.axis_index("c")
        b0 = w * 128
        lane16 = lax.broadcasted_iota(jnp.int32, (16,), 0)
        zeros16 = jnp.zeros((16,), jnp.int32)

        def body(lh, carry):
            pltpu.sync_copy(o3_hbm.at[pl.ds(b0, 128), pl.ds(lh, 1), :],
                            xbuf.at[:, :, pl.ds(0, 128)])
            for h in range(2):
                def rbody(r, c2):
                    col16 = jnp.full((16,), 64 * h + r, jnp.int32)
                    for k in range(8):
                        v = plsc.load_gather(
                            xbuf, [lane16 + 16 * k, zeros16, col16])
                        obuf[pl.ds(r * 128 + 16 * k, 16)] = v
                    return c2
                lax.fori_loop(0, 64, rbody, 0)
                l_out = 2 * lh + h
                for dg in range(8):
                    pltpu.sync_copy(
                        obuf.at[pl.ds(dg * 1024, 1024)],
                        f_hbm.at[pl.ds(((l_out * 8 + dg) * 32 + w) * 1024,
                                       1024)])
            return carry

        lax.fori_loop(0, 100, body, 0)

    return fmt_kernel(o3)


@functools.partial(jax.jit, static_argnums=(2,))
def _emb_gather(idx2d, table, rows_per_w):
    nrows = idx2d.shape[0]
    steps = rows_per_w // K

    mesh = plsc.VectorSubcoreMesh(core_axis_name="c", subcore_axis_name="s")

    @functools.partial(
        pl.kernel,
        mesh=mesh,
        out_type=jax.ShapeDtypeStruct((nrows, RPB, DIM), jnp.float32),
        scratch_types=[
            pltpu.VMEM((rows_per_w, RPB), jnp.int32),
            pltpu.VMEM((NB, K, RPB, DIM), jnp.float32),
            pltpu.SemaphoreType.DMA,
            pltpu.SemaphoreType.DMA,
            pltpu.SemaphoreType.DMA,
        ],
        compiler_params=pltpu.CompilerParams(use_tc_tiling_on_sc=False),
    )
    def gather_kernel(idx_hbm, table_hbm, out_hbm, idx_v, rows_v, gsem,
                      osem0, osem1):
        wid = lax.axis_index("s") * NC + lax.axis_index("c")
        base = wid * rows_per_w
        osems = (osem0, osem1)

        pltpu.sync_copy(idx_hbm.at[pl.ds(base, rows_per_w)], idx_v)

        def fire(s, b):
            return [
                pltpu.async_copy(
                    table_hbm.at[idx_v.at[s * K + j]],
                    rows_v.at[b, j], gsem)
                for j in range(K)
            ]

        def store(s, b):
            pltpu.async_copy(rows_v.at[b], out_hbm.at[pl.ds(base + s * K, K)],
                             osems[b])

        def drain_store(b):
            pltpu.make_async_copy(
                rows_v.at[b], out_hbm.at[pl.ds(base, K)], osems[b]).wait()

        g0 = fire(0, 0)
        g1 = fire(1, 1)
        for c in g0:
            c.wait()
        store(0, 0)
        for c in g1:
            c.wait()
        store(1, 1)

        def body(i, carry):
            s0 = 2 * i
            drain_store(0)
            c0 = fire(s0, 0)
            drain_store(1)
            c1 = fire(s0 + 1, 1)
            for c in c0:
                c.wait()
            store(s0, 0)
            for c in c1:
                c.wait()
            store(s0 + 1, 1)
            return carry

        lax.fori_loop(1, steps // 2, body, 0)

        drain_store(0)
        drain_store(1)

    return gather_kernel(idx2d, table)


def kernel(tokens, emb_weight):
    b, l = tokens.shape
    n = b * l
    vocab, dim = emb_weight.shape
    nrows = n // RPB
    rows_per_w = nrows // NW
    z = _pack_table(emb_weight.T)
    table_lin = z.reshape(vocab, dim)
    idx2d = tokens.reshape(nrows, RPB)
    out = _emb_gather(idx2d, table_lin, rows_per_w)
    o3 = out.reshape(b, l // 2, 2 * DIM)
    f = _fmt_out(o3).reshape(l, 8, b // RPB, 8, RPB)
    return f.transpose(2, 4, 0, 1, 3).reshape(b, l, DIM)


# speedup vs baseline: 1.4681x; 1.4681x over previous
import functools

import jax
import jax.numpy as jnp
from jax import lax
from jax.experimental import pallas as pl
from jax.experimental.pallas import tpu as pltpu
from jax.experimental.pallas import tpu_sc as plsc

DIM = 64
RPB = 128
K = 5
NB = 2
NC = 2
NS = 16
NW = NC * NS
PACK_C = 4096


def _pack_body(t2_ref, z_ref, scr):
    t = t2_ref[...].T
    scr[:, 0:64] = t
    ev = scr[0::2, :]
    od = scr[1::2, :]
    odr = pltpu.roll(od, 64, 1)
    lane = lax.broadcasted_iota(jnp.int32, (PACK_C // 2, 128), 1)
    z_ref[...] = jnp.where(lane < 64, ev, odr)


def _pack_table(t2):
    vocab = t2.shape[1]
    return pl.pallas_call(
        _pack_body,
        grid=(pl.cdiv(vocab, PACK_C),),
        in_specs=[pl.BlockSpec((DIM, PACK_C), lambda j: (0, j))],
        out_specs=pl.BlockSpec((PACK_C // 2, 128), lambda j: (j, 0)),
        out_shape=jax.ShapeDtypeStruct((vocab // 2, 128), jnp.float32),
        scratch_shapes=[pltpu.VMEM((PACK_C, 128), jnp.float32)],
    )(t2)


XW = 131


@jax.jit
def _fmt_out(o3):
    mesh = plsc.VectorSubcoreMesh(core_axis_name="c", subcore_axis_name="s")

    @functools.partial(
        pl.kernel,
        mesh=mesh,
        out_type=jax.ShapeDtypeStruct((200 * 8 * 32 * 8 * 128,), jnp.float32),
        scratch_types=[
            pltpu.VMEM((128, 1, XW), jnp.float32),
            pltpu.VMEM((8192,), jnp.float32),
        ],
        compiler_params=pltpu.CompilerParams(use_tc_tiling_on_sc=False,
                                             needs_layout_passes=False),
    )
    def fmt_kernel(o3_hbm, f_hbm, xbuf, obuf):
        w = lax.axis_index("s") * NC + lax.axis_index("c")
        b0 = w * 128
        lane16 = lax.broadcasted_iota(jnp.int32, (16,), 0)
        zeros16 = jnp.zeros((16,), jnp.int32)

        def body(lh, carry):
            pltpu.sync_copy(o3_hbm.at[pl.ds(b0, 128), pl.ds(lh, 1), :],
                            xbuf.at[:, :, pl.ds(0, 128)])
            for h in range(2):
                def rbody(r, c2):
                    col16 = jnp.full((16,), 64 * h + r, jnp.int32)
                    for k in range(8):
                        v = plsc.load_gather(
                            xbuf, [lane16 + 16 * k, zeros16, col16])
                        obuf[pl.ds(r * 128 + 16 * k, 16)] = v
                    return c2
                lax.fori_loop(0, 64, rbody, 0)
                l_out = 2 * lh + h
                for dg in range(8):
                    pltpu.sync_copy(
                        obuf.at[pl.ds(dg * 1024, 1024)],
                        f_hbm.at[pl.ds(((l_out * 8 + dg) * 32 + w) * 1024,
                                       1024)])
            return carry

        lax.fori_loop(0, 100, body, 0)

    return fmt_kernel(o3)


@functools.partial(jax.jit, static_argnums=(2,))
def _emb_gather(idx2d, table, rows_per_w):
    nrows = idx2d.shape[0]
    steps = rows_per_w // K

    mesh = plsc.VectorSubcoreMesh(core_axis_name="c", subcore_axis_name="s")

    @functools.partial(
        pl.kernel,
        mesh=mesh,
        out_type=jax.ShapeDtypeStruct((nrows, RPB, DIM), jnp.float32),
        scratch_types=[
            pltpu.VMEM((rows_per_w, RPB), jnp.int32),
            pltpu.VMEM((NB, K, RPB, DIM), jnp.float32),
            pltpu.SemaphoreType.DMA,
            pltpu.SemaphoreType.DMA,
            pltpu.SemaphoreType.DMA,
        ],
        compiler_params=pltpu.CompilerParams(use_tc_tiling_on_sc=False),
    )
    def gather_kernel(idx_hbm, table_hbm, out_hbm, idx_v, rows_v, gsem,
                      osem0, osem1):
        wid = lax.axis_index("s") * NC + lax.axis_index("c")
        base = wid * rows_per_w
        osems = (osem0, osem1)

        pltpu.sync_copy(idx_hbm.at[pl.ds(base, rows_per_w)], idx_v)

        def fire(s, b):
            return [
                pltpu.async_copy(
                    table_hbm.at[idx_v.at[s * K + j]],
                    rows_v.at[b, j], gsem)
                for j in range(K)
            ]

        def store(s, b):
            pltpu.async_copy(rows_v.at[b], out_hbm.at[pl.ds(base + s * K, K)],
                             osems[b])

        def drain_store(b):
            pltpu.make_async_copy(
                rows_v.at[b], out_hbm.at[pl.ds(base, K)], osems[b]).wait()

        g0 = fire(0, 0)
        g1 = fire(1, 1)
        for c in g0:
            c.wait()
        store(0, 0)
        for c in g1:
            c.wait()
        store(1, 1)

        def body(i, carry):
            s0 = 2 * i
            drain_store(0)
            c0 = fire(s0, 0)
            drain_store(1)
            c1 = fire(s0 + 1, 1)
            for c in c0:
                c.wait()
            store(s0, 0)
            for c in c1:
                c.wait()
            store(s0 + 1, 1)
            return carry

        lax.fori_loop(1, steps // 2, body, 0)

        drain_store(0)
        drain_store(1)

    return gather_kernel(idx2d, table)


def kernel(tokens, emb_weight):
    b, l = tokens.shape
    n = b * l
    vocab, dim = emb_weight.shape
    nrows = n // RPB
    rows_per_w = nrows // NW
    z = _pack_table(emb_weight.T)
    table_lin = z.reshape(vocab, dim)
    idx2d = tokens.reshape(nrows, RPB)
    out = _emb_gather(idx2d, table_lin, rows_per_w)
    return out.reshape(b, l, DIM)


# speedup vs baseline: 1.5760x; 1.0735x over previous
import functools

import jax
import jax.numpy as jnp
from jax import lax
from jax.experimental import pallas as pl
from jax.experimental.pallas import tpu as pltpu
from jax.experimental.pallas import tpu_sc as plsc

DIM = 64
RPB = 128
K = 5
NB = 2
NC = 2
NS = 16
NW = NC * NS
PACK_C = 8192


def _pack_body(t2_ref, z_ref, scr):
    t = t2_ref[...].T
    scr[:, 0:64] = t
    ev = scr[0::2, :]
    od = scr[1::2, :]
    odr = pltpu.roll(od, 64, 1)
    lane = lax.broadcasted_iota(jnp.int32, (PACK_C // 2, 128), 1)
    z_ref[...] = jnp.where(lane < 64, ev, odr)


def _pack_table(t2):
    vocab = t2.shape[1]
    return pl.pallas_call(
        _pack_body,
        grid=(pl.cdiv(vocab, PACK_C),),
        in_specs=[pl.BlockSpec((DIM, PACK_C), lambda j: (0, j))],
        out_specs=pl.BlockSpec((PACK_C // 2, 128), lambda j: (j, 0)),
        out_shape=jax.ShapeDtypeStruct((vocab // 2, 128), jnp.float32),
        scratch_shapes=[pltpu.VMEM((PACK_C, 128), jnp.float32)],
    )(t2)


XW = 131


@jax.jit
def _fmt_out(o3):
    mesh = plsc.VectorSubcoreMesh(core_axis_name="c", subcore_axis_name="s")

    @functools.partial(
        pl.kernel,
        mesh=mesh,
        out_type=jax.ShapeDtypeStruct((200 * 8 * 32 * 8 * 128,), jnp.float32),
        scratch_types=[
            pltpu.VMEM((128, 1, XW), jnp.float32),
            pltpu.VMEM((8192,), jnp.float32),
        ],
        compiler_params=pltpu.CompilerParams(use_tc_tiling_on_sc=False,
                                             needs_layout_passes=False),
    )
    def fmt_kernel(o3_hbm, f_hbm, xbuf, obuf):
        w = lax.axis_index("s") * NC + lax.axis_index("c")
        b0 = w * 128
        lane16 = lax.broadcasted_iota(jnp.int32, (16,), 0)
        zeros16 = jnp.zeros((16,), jnp.int32)

        def body(lh, carry):
            pltpu.sync_copy(o3_hbm.at[pl.ds(b0, 128), pl.ds(lh, 1), :],
                            xbuf.at[:, :, pl.ds(0, 128)])
            for h in range(2):
                def rbody(r, c2):
                    col16 = jnp.full((16,), 64 * h + r, jnp.int32)
                    for k in range(8):
                        v = plsc.load_gather(
                            xbuf, [lane16 + 16 * k, zeros16, col16])
                        obuf[pl.ds(r * 128 + 16 * k, 16)] = v
                    return c2
                lax.fori_loop(0, 64, rbody, 0)
                l_out = 2 * lh + h
                for dg in range(8):
                    pltpu.sync_copy(
                        obuf.at[pl.ds(dg * 1024, 1024)],
                        f_hbm.at[pl.ds(((l_out * 8 + dg) * 32 + w) * 1024,
                                       1024)])
            return carry

        lax.fori_loop(0, 100, body, 0)

    return fmt_kernel(o3)


@functools.partial(jax.jit, static_argnums=(2,))
def _emb_gather(idx2d, table, rows_per_w):
    nrows = idx2d.shape[0]
    steps = rows_per_w // K

    mesh = plsc.VectorSubcoreMesh(core_axis_name="c", subcore_axis_name="s")

    @functools.partial(
        pl.kernel,
        mesh=mesh,
        out_type=jax.ShapeDtypeStruct((nrows, RPB, DIM), jnp.float32),
        scratch_types=[
            pltpu.VMEM((rows_per_w, RPB), jnp.int32),
            pltpu.VMEM((NB, K, RPB, DIM), jnp.float32),
            pltpu.SemaphoreType.DMA,
            pltpu.SemaphoreType.DMA,
            pltpu.SemaphoreType.DMA,
        ],
        compiler_params=pltpu.CompilerParams(use_tc_tiling_on_sc=False),
    )
    def gather_kernel(idx_hbm, table_hbm, out_hbm, idx_v, rows_v, gsem,
                      osem0, osem1):
        wid = lax.axis_index("s") * NC + lax.axis_index("c")
        base = wid * rows_per_w
        osems = (osem0, osem1)

        pltpu.sync_copy(idx_hbm.at[pl.ds(base, rows_per_w)], idx_v)

        def fire(s, b):
            return [
                pltpu.async_copy(
                    table_hbm.at[idx_v.at[s * K + j]],
                    rows_v.at[b, j], gsem)
                for j in range(K)
            ]

        def store(s, b):
            pltpu.async_copy(rows_v.at[b], out_hbm.at[pl.ds(base + s * K, K)],
                             osems[b])

        def drain_store(b):
            pltpu.make_async_copy(
                rows_v.at[b], out_hbm.at[pl.ds(base, K)], osems[b]).wait()

        g0 = fire(0, 0)
        g1 = fire(1, 1)
        for c in g0:
            c.wait()
        store(0, 0)
        for c in g1:
            c.wait()
        store(1, 1)

        def body(i, carry):
            s0 = 2 * i
            drain_store(0)
            c0 = fire(s0, 0)
            drain_store(1)
            c1 = fire(s0 + 1, 1)
            for c in c0:
                c.wait()
            store(s0, 0)
            for c in c1:
                c.wait()
            store(s0 + 1, 1)
            return carry

        lax.fori_loop(1, steps // 2, body, 0)

        drain_store(0)
        drain_store(1)

    return gather_kernel(idx2d, table)


def kernel(tokens, emb_weight):
    b, l = tokens.shape
    n = b * l
    vocab, dim = emb_weight.shape
    nrows = n // RPB
    rows_per_w = nrows // NW
    z = _pack_table(emb_weight.T)
    table_lin = z.reshape(vocab, dim)
    idx2d = tokens.reshape(nrows, RPB)
    out = _emb_gather(idx2d, table_lin, rows_per_w)
    return out.reshape(b, l, DIM)


# speedup vs baseline: 1.6379x; 1.0393x over previous
import functools

import jax
import jax.numpy as jnp
from jax import lax
from jax.experimental import pallas as pl
from jax.experimental.pallas import tpu as pltpu
from jax.experimental.pallas import tpu_sc as plsc

DIM = 64
RPB = 128
K = 5
NB = 2
NC = 2
NS = 16
NW = NC * NS
PACK_C = 16384


def _pack_body(t2_ref, z_ref, scr):
    t = t2_ref[...].T
    scr[:, 0:64] = t
    ev = scr[0::2, :]
    od = scr[1::2, :]
    odr = pltpu.roll(od, 64, 1)
    lane = lax.broadcasted_iota(jnp.int32, (PACK_C // 2, 128), 1)
    z_ref[...] = jnp.where(lane < 64, ev, odr)


def _pack_table(t2):
    vocab = t2.shape[1]
    return pl.pallas_call(
        _pack_body,
        grid=(pl.cdiv(vocab, PACK_C),),
        in_specs=[pl.BlockSpec((DIM, PACK_C), lambda j: (0, j))],
        out_specs=pl.BlockSpec((PACK_C // 2, 128), lambda j: (j, 0)),
        out_shape=jax.ShapeDtypeStruct((vocab // 2, 128), jnp.float32),
        scratch_shapes=[pltpu.VMEM((PACK_C, 128), jnp.float32)],
    )(t2)


XW = 131


@jax.jit
def _fmt_out(o3):
    mesh = plsc.VectorSubcoreMesh(core_axis_name="c", subcore_axis_name="s")

    @functools.partial(
        pl.kernel,
        mesh=mesh,
        out_type=jax.ShapeDtypeStruct((200 * 8 * 32 * 8 * 128,), jnp.float32),
        scratch_types=[
            pltpu.VMEM((128, 1, XW), jnp.float32),
            pltpu.VMEM((8192,), jnp.float32),
        ],
        compiler_params=pltpu.CompilerParams(use_tc_tiling_on_sc=False,
                                             needs_layout_passes=False),
    )
    def fmt_kernel(o3_hbm, f_hbm, xbuf, obuf):
        w = lax.axis_index("s") * NC + lax.axis_index("c")
        b0 = w * 128
        lane16 = lax.broadcasted_iota(jnp.int32, (16,), 0)
        zeros16 = jnp.zeros((16,), jnp.int32)

        def body(lh, carry):
            pltpu.sync_copy(o3_hbm.at[pl.ds(b0, 128), pl.ds(lh, 1), :],
                            xbuf.at[:, :, pl.ds(0, 128)])
            for h in range(2):
                def rbody(r, c2):
                    col16 = jnp.full((16,), 64 * h + r, jnp.int32)
                    for k in range(8):
                        v = plsc.load_gather(
                            xbuf, [lane16 + 16 * k, zeros16, col16])
                        obuf[pl.ds(r * 128 + 16 * k, 16)] = v
                    return c2
                lax.fori_loop(0, 64, rbody, 0)
                l_out = 2 * lh + h
                for dg in range(8):
                    pltpu.sync_copy(
                        obuf.at[pl.ds(dg * 1024, 1024)],
                        f_hbm.at[pl.ds(((l_out * 8 + dg) * 32 + w) * 1024,
                                       1024)])
            return carry

        lax.fori_loop(0, 100, body, 0)

    return fmt_kernel(o3)


@functools.partial(jax.jit, static_argnums=(2,))
def _emb_gather(idx2d, table, rows_per_w):
    nrows = idx2d.shape[0]
    steps = rows_per_w // K

    mesh = plsc.VectorSubcoreMesh(core_axis_name="c", subcore_axis_name="s")

    @functools.partial(
        pl.kernel,
        mesh=mesh,
        out_type=jax.ShapeDtypeStruct((nrows, RPB, DIM), jnp.float32),
        scratch_types=[
            pltpu.VMEM((rows_per_w, RPB), jnp.int32),
            pltpu.VMEM((NB, K, RPB, DIM), jnp.float32),
            pltpu.SemaphoreType.DMA,
            pltpu.SemaphoreType.DMA,
            pltpu.SemaphoreType.DMA,
        ],
        compiler_params=pltpu.CompilerParams(use_tc_tiling_on_sc=False),
    )
    def gather_kernel(idx_hbm, table_hbm, out_hbm, idx_v, rows_v, gsem,
                      osem0, osem1):
        wid = lax.axis_index("s") * NC + lax.axis_index("c")
        base = wid * rows_per_w
        osems = (osem0, osem1)

        pltpu.sync_copy(idx_hbm.at[pl.ds(base, rows_per_w)], idx_v)

        def fire(s, b):
            return [
                pltpu.async_copy(
                    table_hbm.at[idx_v.at[s * K + j]],
                    rows_v.at[b, j], gsem)
                for j in range(K)
            ]

        def store(s, b):
            pltpu.async_copy(rows_v.at[b], out_hbm.at[pl.ds(base + s * K, K)],
                             osems[b])

        def drain_store(b):
            pltpu.make_async_copy(
                rows_v.at[b], out_hbm.at[pl.ds(base, K)], osems[b]).wait()

        g0 = fire(0, 0)
        g1 = fire(1, 1)
        for c in g0:
            c.wait()
        store(0, 0)
        for c in g1:
            c.wait()
        store(1, 1)

        def body(i, carry):
            s0 = 2 * i
            drain_store(0)
            c0 = fire(s0, 0)
            drain_store(1)
            c1 = fire(s0 + 1, 1)
            for c in c0:
                c.wait()
            store(s0, 0)
            for c in c1:
                c.wait()
            store(s0 + 1, 1)
            return carry

        lax.fori_loop(1, steps // 2, body, 0)

        drain_store(0)
        drain_store(1)

    return gather_kernel(idx2d, table)


def kernel(tokens, emb_weight):
    b, l = tokens.shape
    n = b * l
    vocab, dim = emb_weight.shape
    nrows = n // RPB
    rows_per_w = nrows // NW
    z = _pack_table(emb_weight.T)
    table_lin = z.reshape(vocab, dim)
    idx2d = tokens.reshape(nrows, RPB)
    out = _emb_gather(idx2d, table_lin, rows_per_w)
    return out.reshape(b, l, DIM)


# speedup vs baseline: 1.6633x; 1.0155x over previous
import functools

import jax
import jax.numpy as jnp
from jax import lax
from jax.experimental import pallas as pl
from jax.experimental.pallas import tpu as pltpu
from jax.experimental.pallas import tpu_sc as plsc

DIM = 64
RPB = 128
K = 5
NB = 2
NC = 2
NS = 16
NW = NC * NS
PACK_C = 32768


def _pack_body(t2_ref, z_ref, scr):
    t = t2_ref[...].T
    scr[:, 0:64] = t
    ev = scr[0::2, :]
    od = scr[1::2, :]
    odr = pltpu.roll(od, 64, 1)
    lane = lax.broadcasted_iota(jnp.int32, (PACK_C // 2, 128), 1)
    z_ref[...] = jnp.where(lane < 64, ev, odr)


def _pack_table(t2):
    vocab = t2.shape[1]
    return pl.pallas_call(
        _pack_body,
        grid=(pl.cdiv(vocab, PACK_C),),
        in_specs=[pl.BlockSpec((DIM, PACK_C), lambda j: (0, j))],
        out_specs=pl.BlockSpec((PACK_C // 2, 128), lambda j: (j, 0)),
        out_shape=jax.ShapeDtypeStruct((vocab // 2, 128), jnp.float32),
        scratch_shapes=[pltpu.VMEM((PACK_C, 128), jnp.float32)],
    )(t2)


XW = 131


@jax.jit
def _fmt_out(o3):
    mesh = plsc.VectorSubcoreMesh(core_axis_name="c", subcore_axis_name="s")

    @functools.partial(
        pl.kernel,
        mesh=mesh,
        out_type=jax.ShapeDtypeStruct((200 * 8 * 32 * 8 * 128,), jnp.float32),
        scratch_types=[
            pltpu.VMEM((128, 1, XW), jnp.float32),
            pltpu.VMEM((8192,), jnp.float32),
        ],
        compiler_params=pltpu.CompilerParams(use_tc_tiling_on_sc=False,
                                             needs_layout_passes=False),
    )
    def fmt_kernel(o3_hbm, f_hbm, xbuf, obuf):
        w = lax.axis_index("s") * NC + lax.axis_index("c")
        b0 = w * 128
        lane16 = lax.broadcasted_iota(jnp.int32, (16,), 0)
        zeros16 = jnp.zeros((16,), jnp.int32)

        def body(lh, carry):
            pltpu.sync_copy(o3_hbm.at[pl.ds(b0, 128), pl.ds(lh, 1), :],
                            xbuf.at[:, :, pl.ds(0, 128)])
            for h in range(2):
                def rbody(r, c2):
                    col16 = jnp.full((16,), 64 * h + r, jnp.int32)
                    for k in range(8):
                        v = plsc.load_gather(
                            xbuf, [lane16 + 16 * k, zeros16, col16])
                        obuf[pl.ds(r * 128 + 16 * k, 16)] = v
                    return c2
                lax.fori_loop(0, 64, rbody, 0)
                l_out = 2 * lh + h
                for dg in range(8):
                    pltpu.sync_copy(
                        obuf.at[pl.ds(dg * 1024, 1024)],
                        f_hbm.at[pl.ds(((l_out * 8 + dg) * 32 + w) * 1024,
                                       1024)])
            return carry

        lax.fori_loop(0, 100, body, 0)

    return fmt_kernel(o3)


@functools.partial(jax.jit, static_argnums=(2,))
def _emb_gather(idx2d, table, rows_per_w):
    nrows = idx2d.shape[0]
    steps = rows_per_w // K

    mesh = plsc.VectorSubcoreMesh(core_axis_name="c", subcore_axis_name="s")

    @functools.partial(
        pl.kernel,
        mesh=mesh,
        out_type=jax.ShapeDtypeStruct((nrows, RPB, DIM), jnp.float32),
        scratch_types=[
            pltpu.VMEM((rows_per_w, RPB), jnp.int32),
            pltpu.VMEM((NB, K, RPB, DIM), jnp.float32),
            pltpu.SemaphoreType.DMA,
            pltpu.SemaphoreType.DMA,
            pltpu.SemaphoreType.DMA,
        ],
        compiler_params=pltpu.CompilerParams(use_tc_tiling_on_sc=False),
    )
    def gather_kernel(idx_hbm, table_hbm, out_hbm, idx_v, rows_v, gsem,
                      osem0, osem1):
        wid = lax.axis_index("s") * NC + lax.axis_index("c")
        base = wid * rows_per_w
        osems = (osem0, osem1)

        pltpu.sync_copy(idx_hbm.at[pl.ds(base, rows_per_w)], idx_v)

        def fire(s, b):
            return [
                pltpu.async_copy(
                    table_hbm.at[idx_v.at[s * K + j]],
                    rows_v.at[b, j], gsem)
                for j in range(K)
            ]

        def store(s, b):
            pltpu.async_copy(rows_v.at[b], out_hbm.at[pl.ds(base + s * K, K)],
                             osems[b])

        def drain_store(b):
            pltpu.make_async_copy(
                rows_v.at[b], out_hbm.at[pl.ds(base, K)], osems[b]).wait()

        g0 = fire(0, 0)
        g1 = fire(1, 1)
        for c in g0:
            c.wait()
        store(0, 0)
        for c in g1:
            c.wait()
        store(1, 1)

        def body(i, carry):
            s0 = 2 * i
            drain_store(0)
            c0 = fire(s0, 0)
            drain_store(1)
            c1 = fire(s0 + 1, 1)
            for c in c0:
                c.wait()
            store(s0, 0)
            for c in c1:
                c.wait()
            store(s0 + 1, 1)
            return carry

        lax.fori_loop(1, steps // 2, body, 0)

        drain_store(0)
        drain_store(1)

    return gather_kernel(idx2d, table)


def kernel(tokens, emb_weight):
    b, l = tokens.shape
    n = b * l
    vocab, dim = emb_weight.shape
    nrows = n // RPB
    rows_per_w = nrows // NW
    z = _pack_table(emb_weight.T)
    table_lin = z.reshape(vocab, dim)
    idx2d = tokens.reshape(nrows, RPB)
    out = _emb_gather(idx2d, table_lin, rows_per_w)
    return out.reshape(b, l, DIM)
